# baseline (speedup 1.0000x reference)
import jax
import jax.numpy as jnp
from jax.experimental import pallas as pl
from jax.experimental.pallas import tpu as pltpu

_IB = 4


def _tdot(lhs, rhs):
    return jax.lax.dot_general(lhs, rhs, (((0,), (0,)), ((), ())),
                               preferred_element_type=jnp.float32)


def _fused_kernel(pool_ref, ind_ref, ind2_ref, att_ref, mask_ref,
                  fw1_ref, fb1_ref, fw2_ref, fb2_ref,
                  pw1_ref, pb1_ref, pw2_ref, pb2_ref,
                  spos_ref, sneg_ref, lpart_ref, fc_ref, af_ref, amask_ref):
    B2, N, _ = pool_ref.shape
    IB, O, L = att_ref.shape
    M = B2 // (2 * IB)
    S = 2 * M

    indb = ind_ref[...]
    att3 = att_ref[...]
    iota_f = jax.lax.broadcasted_iota(jnp.int32, (O, S * N), 0)
    g_list = []
    for i in range(IB):
        oh_i = jnp.where(ind2_ref[i] == iota_f, 1.0, 0.0)
        g_list.append(jax.lax.dot_general(
            oh_i, att3[i],
            dimension_numbers=(((0,), (0,)), ((), ())),
            preferred_element_type=jnp.float32))
    gathered = jnp.concatenate(g_list, axis=0).reshape(B2, N, L)

    clean = jax.lax.dot_general(
        pool_ref[...], gathered,
        dimension_numbers=(((2,), (1,)), ((0,), (0,))),
        preferred_element_type=jnp.float32)

    max_feat = jnp.max(clean, axis=1)
    mask = mask_ref[...]
    mask_sum = jnp.sum(mask, axis=1, keepdims=True)
    inv = pl.reciprocal(jnp.maximum(mask_sum, 1.0), approx=True)
    mean_feat = jnp.sum(clean, axis=1) * inv

    w1 = fw1_ref[...]
    h = (jnp.dot(max_feat, w1[:L, :], preferred_element_type=jnp.float32)
         + jnp.dot(mean_feat, w1[L:, :], preferred_element_type=jnp.float32)
         + fb1_ref[...])
    h = jnp.maximum(h, 0.0)
    z = jnp.sum(h * fw2_ref[...], axis=-1, keepdims=True) + fb2_ref[...]
    p = jax.nn.sigmoid(z)

    row = jax.lax.broadcasted_iota(jnp.int32, (B2, 1), 0)
    t = jnp.where(row % S < M, 1.0, 0.0)
    log_p = jnp.maximum(jnp.log(p), -100.0)
    log_1mp = jnp.maximum(jnp.log(1.0 - p), -100.0)
    lv = -(t * log_p + (1.0 - t) * log_1mp)

    m_idx = jax.lax.broadcasted_iota(jnp.int32, (M, 1), 0)
    iota_on = jax.lax.broadcasted_iota(jnp.int32, (O, N), 0)
    for i in range(IB):
        base = i * S
        spos_ref[i] = p[base:base + M]
        sneg_ref[i] = p[base + M:base + S]
        lpart_ref[i] = jnp.sum(lv[base:base + S], keepdims=True)

        p_pos = p[base:base + M]
        am = jnp.min(jnp.where(p_pos == jnp.max(p_pos), m_idx, M))
        sel = m_idx == am

        smax = jnp.sum(jnp.where(sel, max_feat[base:base + M, :], 0.0),
                       axis=0, keepdims=True)
        smean = jnp.sum(jnp.where(sel, mean_feat[base:base + M, :], 0.0),
                        axis=0, keepdims=True)
        x = jnp.concatenate([smax, smean], axis=1)
        hh = (jnp.dot(x, pw1_ref[...], preferred_element_type=jnp.float32)
              + pb1_ref[...])
        fc_ref[i] = (jnp.dot(hh, pw2_ref[...], preferred_element_type=jnp.float32)
                     + pb2_ref[...])

        ind_pos = indb[base:base + M, :]
        rowsel = jnp.sum(jnp.where(sel, ind_pos, 0), axis=0, keepdims=True)
        ohT = (iota_on == rowsel).astype(jnp.bfloat16)
        a = att3[i]
        a1 = a.astype(jnp.bfloat16)
        r1 = a - a1.astype(jnp.float32)
        a2 = r1.astype(jnp.bfloat16)
        a3 = (r1 - a2.astype(jnp.float32)).astype(jnp.bfloat16)
        af_ref[i] = (_tdot(ohT, a1) + _tdot(ohT, a2)) + _tdot(ohT, a3)

        mask_pos = mask[base:base + M, :]
        amask_ref[i] = jnp.sum(jnp.where(sel, mask_pos, 0.0),
                               axis=0, keepdims=True)


def kernel(fc_w1, fc_b1, fc_w2, fc_b2, proj_w1, proj_b1, proj_w2, proj_b2,
           gpn_obj_ind, gpn_pred_ind, gpn_nrel_ind, gpn_pool_mtx,
           att_feats, x_pred, fc_feats, att_masks):
    b, two, M, N, _ = gpn_pool_mtx.shape
    O, L = att_feats.shape[1], att_feats.shape[2]
    hid = fc_w1.shape[-1]
    G = two * b * M
    IB = _IB
    R = b * two * M
    B2 = IB * two * M

    pool_r = gpn_pool_mtx.reshape(R, N, N)
    ind_r = gpn_obj_ind.reshape(R, N)
    ind2_r = gpn_obj_ind.reshape(b, 1, two * M * N)
    mask_r = att_masks.reshape(R, N)

    spos, sneg, lparts, fc, af, amask = pl.pallas_call(
        _fused_kernel,
        out_shape=(
            jax.ShapeDtypeStruct((b, M, 1), jnp.float32),
            jax.ShapeDtypeStruct((b, M, 1), jnp.float32),
            jax.ShapeDtypeStruct((b, 1, 1), jnp.float32),
            jax.ShapeDtypeStruct((b, 1, 2 * L), jnp.float32),
            jax.ShapeDtypeStruct((b, N, L), jnp.float32),
            jax.ShapeDtypeStruct((b, 1, N), jnp.float32),
        ),
        grid=(b // IB,),
        in_specs=[
            pl.BlockSpec((B2, N, N), lambda g: (g, 0, 0)),
            pl.BlockSpec((B2, N), lambda g: (g, 0)),
            pl.BlockSpec((IB, 1, two * M * N), lambda g: (g, 0, 0)),
            pl.BlockSpec((IB, O, L), lambda g: (g, 0, 0)),
            pl.BlockSpec((B2, N), lambda g: (g, 0)),
            pl.BlockSpec((2 * L, hid), lambda g: (0, 0)),
            pl.BlockSpec((1, hid), lambda g: (0, 0)),
            pl.BlockSpec((1, hid), lambda g: (0, 0)),
            pl.BlockSpec((1, 1), lambda g: (0, 0)),
            pl.BlockSpec((2 * L, hid), lambda g: (0, 0)),
            pl.BlockSpec((1, hid), lambda g: (0, 0)),
            pl.BlockSpec((hid, 2 * L), lambda g: (0, 0)),
            pl.BlockSpec((1, 2 * L), lambda g: (0, 0)),
        ],
        out_specs=(
            pl.BlockSpec((IB, M, 1), lambda g: (g, 0, 0)),
            pl.BlockSpec((IB, M, 1), lambda g: (g, 0, 0)),
            pl.BlockSpec((IB, 1, 1), lambda g: (g, 0, 0)),
            pl.BlockSpec((IB, 1, 2 * L), lambda g: (g, 0, 0)),
            pl.BlockSpec((IB, N, L), lambda g: (g, 0, 0)),
            pl.BlockSpec((IB, 1, N), lambda g: (g, 0, 0)),
        ),
        compiler_params=pltpu.CompilerParams(
            dimension_semantics=("parallel",)),
    )(pool_r, ind_r, ind2_r, att_feats, mask_r,
      fc_w1, fc_b1, fc_w2, fc_b2, proj_w1, proj_b1, proj_w2, proj_b2)

    gpn_loss = jnp.sum(lparts) / G
    subgraph_score = jnp.concatenate(
        [spos.reshape(b * M, 1), sneg.reshape(b * M, 1)], axis=0)
    fc_feats_out = fc.reshape(b, 2 * L)
    att_masks_out = amask.reshape(b, N)

    return gpn_loss, subgraph_score, af, fc_feats_out, att_masks_out


# speedup vs baseline: 1.1175x; 1.1175x over previous
import jax
import jax.numpy as jnp
from jax.experimental import pallas as pl
from jax.experimental.pallas import tpu as pltpu

_IB = 8


def _tdot(lhs, rhs):
    return jax.lax.dot_general(lhs, rhs, (((0,), (0,)), ((), ())),
                               preferred_element_type=jnp.float32)


def _fused_kernel(pool_ref, ind_ref, ind2_ref, att_ref, mask_ref,
                  fw1_ref, fb1_ref, fw2_ref, fb2_ref,
                  pw1_ref, pb1_ref, pw2_ref, pb2_ref,
                  spos_ref, sneg_ref, lpart_ref, fc_ref, af_ref, amask_ref):
    B2, N, _ = pool_ref.shape
    IB, O, L = att_ref.shape
    M = B2 // (2 * IB)
    S = 2 * M

    indb = ind_ref[...]
    att3 = att_ref[...]
    iota_f = jax.lax.broadcasted_iota(jnp.int32, (O, S * N), 0)
    g_list = []
    for i in range(IB):
        oh_i = jnp.where(ind2_ref[i] == iota_f, 1.0, 0.0)
        g_list.append(jax.lax.dot_general(
            oh_i, att3[i],
            dimension_numbers=(((0,), (0,)), ((), ())),
            preferred_element_type=jnp.float32))
    gathered = jnp.concatenate(g_list, axis=0).reshape(B2, N, L)

    clean = jax.lax.dot_general(
        pool_ref[...], gathered,
        dimension_numbers=(((2,), (1,)), ((0,), (0,))),
        preferred_element_type=jnp.float32)

    max_feat = jnp.max(clean, axis=1)
    mask = mask_ref[...]
    mask_sum = jnp.sum(mask, axis=1, keepdims=True)
    inv = pl.reciprocal(jnp.maximum(mask_sum, 1.0), approx=True)
    mean_feat = jnp.sum(clean, axis=1) * inv

    w1 = fw1_ref[...]
    h = (jnp.dot(max_feat, w1[:L, :], preferred_element_type=jnp.float32)
         + jnp.dot(mean_feat, w1[L:, :], preferred_element_type=jnp.float32)
         + fb1_ref[...])
    h = jnp.maximum(h, 0.0)
    z = jnp.sum(h * fw2_ref[...], axis=-1, keepdims=True) + fb2_ref[...]
    p = jax.nn.sigmoid(z)

    row = jax.lax.broadcasted_iota(jnp.int32, (B2, 1), 0)
    t = jnp.where(row % S < M, 1.0, 0.0)
    log_p = jnp.maximum(jnp.log(p), -100.0)
    log_1mp = jnp.maximum(jnp.log(1.0 - p), -100.0)
    lv = -(t * log_p + (1.0 - t) * log_1mp)

    m_idx = jax.lax.broadcasted_iota(jnp.int32, (M, 1), 0)
    iota_on = jax.lax.broadcasted_iota(jnp.int32, (O, N), 0)
    for i in range(IB):
        base = i * S
        spos_ref[i] = p[base:base + M]
        sneg_ref[i] = p[base + M:base + S]
        lpart_ref[i] = jnp.sum(lv[base:base + S], keepdims=True)

        p_pos = p[base:base + M]
        am = jnp.min(jnp.where(p_pos == jnp.max(p_pos), m_idx, M))
        sel = m_idx == am

        smax = jnp.sum(jnp.where(sel, max_feat[base:base + M, :], 0.0),
                       axis=0, keepdims=True)
        smean = jnp.sum(jnp.where(sel, mean_feat[base:base + M, :], 0.0),
                        axis=0, keepdims=True)
        x = jnp.concatenate([smax, smean], axis=1)
        hh = (jnp.dot(x, pw1_ref[...], preferred_element_type=jnp.float32)
              + pb1_ref[...])
        fc_ref[i] = (jnp.dot(hh, pw2_ref[...], preferred_element_type=jnp.float32)
                     + pb2_ref[...])

        ind_pos = indb[base:base + M, :]
        rowsel = jnp.sum(jnp.where(sel, ind_pos, 0), axis=0, keepdims=True)
        ohT = (iota_on == rowsel).astype(jnp.bfloat16)
        a = att3[i]
        a1 = a.astype(jnp.bfloat16)
        r1 = a - a1.astype(jnp.float32)
        a2 = r1.astype(jnp.bfloat16)
        a3 = (r1 - a2.astype(jnp.float32)).astype(jnp.bfloat16)
        af_ref[i] = (_tdot(ohT, a1) + _tdot(ohT, a2)) + _tdot(ohT, a3)

        mask_pos = mask[base:base + M, :]
        amask_ref[i] = jnp.sum(jnp.where(sel, mask_pos, 0.0),
                               axis=0, keepdims=True)


def kernel(fc_w1, fc_b1, fc_w2, fc_b2, proj_w1, proj_b1, proj_w2, proj_b2,
           gpn_obj_ind, gpn_pred_ind, gpn_nrel_ind, gpn_pool_mtx,
           att_feats, x_pred, fc_feats, att_masks):
    b, two, M, N, _ = gpn_pool_mtx.shape
    O, L = att_feats.shape[1], att_feats.shape[2]
    hid = fc_w1.shape[-1]
    G = two * b * M
    IB = _IB
    R = b * two * M
    B2 = IB * two * M

    pool_r = gpn_pool_mtx.reshape(R, N, N)
    ind_r = gpn_obj_ind.reshape(R, N)
    ind2_r = gpn_obj_ind.reshape(b, 1, two * M * N)
    mask_r = att_masks.reshape(R, N)

    spos, sneg, lparts, fc, af, amask = pl.pallas_call(
        _fused_kernel,
        out_shape=(
            jax.ShapeDtypeStruct((b, M, 1), jnp.float32),
            jax.ShapeDtypeStruct((b, M, 1), jnp.float32),
            jax.ShapeDtypeStruct((b, 1, 1), jnp.float32),
            jax.ShapeDtypeStruct((b, 1, 2 * L), jnp.float32),
            jax.ShapeDtypeStruct((b, N, L), jnp.float32),
            jax.ShapeDtypeStruct((b, 1, N), jnp.float32),
        ),
        grid=(b // IB,),
        in_specs=[
            pl.BlockSpec((B2, N, N), lambda g: (g, 0, 0)),
            pl.BlockSpec((B2, N), lambda g: (g, 0)),
            pl.BlockSpec((IB, 1, two * M * N), lambda g: (g, 0, 0)),
            pl.BlockSpec((IB, O, L), lambda g: (g, 0, 0)),
            pl.BlockSpec((B2, N), lambda g: (g, 0)),
            pl.BlockSpec((2 * L, hid), lambda g: (0, 0)),
            pl.BlockSpec((1, hid), lambda g: (0, 0)),
            pl.BlockSpec((1, hid), lambda g: (0, 0)),
            pl.BlockSpec((1, 1), lambda g: (0, 0)),
            pl.BlockSpec((2 * L, hid), lambda g: (0, 0)),
            pl.BlockSpec((1, hid), lambda g: (0, 0)),
            pl.BlockSpec((hid, 2 * L), lambda g: (0, 0)),
            pl.BlockSpec((1, 2 * L), lambda g: (0, 0)),
        ],
        out_specs=(
            pl.BlockSpec((IB, M, 1), lambda g: (g, 0, 0)),
            pl.BlockSpec((IB, M, 1), lambda g: (g, 0, 0)),
            pl.BlockSpec((IB, 1, 1), lambda g: (g, 0, 0)),
            pl.BlockSpec((IB, 1, 2 * L), lambda g: (g, 0, 0)),
            pl.BlockSpec((IB, N, L), lambda g: (g, 0, 0)),
            pl.BlockSpec((IB, 1, N), lambda g: (g, 0, 0)),
        ),
        compiler_params=pltpu.CompilerParams(
            dimension_semantics=("parallel",)),
    )(pool_r, ind_r, ind2_r, att_feats, mask_r,
      fc_w1, fc_b1, fc_w2, fc_b2, proj_w1, proj_b1, proj_w2, proj_b2)

    gpn_loss = jnp.sum(lparts) / G
    subgraph_score = jnp.concatenate(
        [spos.reshape(b * M, 1), sneg.reshape(b * M, 1)], axis=0)
    fc_feats_out = fc.reshape(b, 2 * L)
    att_masks_out = amask.reshape(b, N)

    return gpn_loss, subgraph_score, af, fc_feats_out, att_masks_out


# speedup vs baseline: 1.1358x; 1.0163x over previous
import jax
import jax.numpy as jnp
from jax.experimental import pallas as pl
from jax.experimental.pallas import tpu as pltpu

_IB = 8


def _tdot(lhs, rhs):
    return jax.lax.dot_general(lhs, rhs, (((0,), (0,)), ((), ())),
                               preferred_element_type=jnp.float32)


def _fused_kernel(pool_ref, ind_ref, ind2_ref, att_ref, mask_ref,
                  fw1_ref, fb1_ref, fw2_ref, fb2_ref,
                  pw1_ref, pb1_ref, pw2_ref, pb2_ref,
                  score_ref, lpart_ref, fc_ref, af_ref, amask_ref):
    B2, N, _ = pool_ref.shape
    IB, O, L = att_ref.shape
    M = B2 // (2 * IB)
    S = 2 * M

    indb = ind_ref[...]
    att3 = att_ref[...]
    iota_f = jax.lax.broadcasted_iota(jnp.int32, (O, S * N), 0)
    g_list = []
    for i in range(IB):
        oh_i = jnp.where(ind2_ref[i] == iota_f, 1.0, 0.0)
        g_list.append(jax.lax.dot_general(
            oh_i, att3[i],
            dimension_numbers=(((0,), (0,)), ((), ())),
            preferred_element_type=jnp.float32))
    gathered = jnp.concatenate(g_list, axis=0).reshape(B2, N, L)

    clean = jax.lax.dot_general(
        pool_ref[...], gathered,
        dimension_numbers=(((2,), (1,)), ((0,), (0,))),
        preferred_element_type=jnp.float32)

    max_feat = jnp.max(clean, axis=1)
    mask = mask_ref[...]
    mask_sum = jnp.sum(mask, axis=1, keepdims=True)
    inv = pl.reciprocal(jnp.maximum(mask_sum, 1.0), approx=True)
    mean_feat = jnp.sum(clean, axis=1) * inv

    w1 = fw1_ref[...]
    h = (jnp.dot(max_feat, w1[:L, :], preferred_element_type=jnp.float32)
         + jnp.dot(mean_feat, w1[L:, :], preferred_element_type=jnp.float32)
         + fb1_ref[...])
    h = jnp.maximum(h, 0.0)
    z = jnp.sum(h * fw2_ref[...], axis=-1, keepdims=True) + fb2_ref[...]
    p = jax.nn.sigmoid(z)

    row = jax.lax.broadcasted_iota(jnp.int32, (B2, 1), 0)
    t = jnp.where(row % S < M, 1.0, 0.0)
    log_p = jnp.maximum(jnp.log(p), -100.0)
    log_1mp = jnp.maximum(jnp.log(1.0 - p), -100.0)
    lv = -(t * log_p + (1.0 - t) * log_1mp)

    m_idx = jax.lax.broadcasted_iota(jnp.int32, (M, 1), 0)
    iota_on = jax.lax.broadcasted_iota(jnp.int32, (O, N), 0)
    for i in range(IB):
        base = i * S
        score_ref[0, i] = p[base:base + M]
        score_ref[1, i] = p[base + M:base + S]
        lpart_ref[i] = jnp.sum(lv[base:base + S], keepdims=True)

        p_pos = p[base:base + M]
        am = jnp.min(jnp.where(p_pos == jnp.max(p_pos), m_idx, M))
        sel = m_idx == am

        smax = jnp.sum(jnp.where(sel, max_feat[base:base + M, :], 0.0),
                       axis=0, keepdims=True)
        smean = jnp.sum(jnp.where(sel, mean_feat[base:base + M, :], 0.0),
                        axis=0, keepdims=True)
        x = jnp.concatenate([smax, smean], axis=1)
        hh = (jnp.dot(x, pw1_ref[...], preferred_element_type=jnp.float32)
              + pb1_ref[...])
        fc_ref[i] = (jnp.dot(hh, pw2_ref[...], preferred_element_type=jnp.float32)
                     + pb2_ref[...])

        ind_pos = indb[base:base + M, :]
        rowsel = jnp.sum(jnp.where(sel, ind_pos, 0), axis=0, keepdims=True)
        ohT = (iota_on == rowsel).astype(jnp.bfloat16)
        a = att3[i]
        a1 = a.astype(jnp.bfloat16)
        r1 = a - a1.astype(jnp.float32)
        a2 = r1.astype(jnp.bfloat16)
        a3 = (r1 - a2.astype(jnp.float32)).astype(jnp.bfloat16)
        af_ref[i] = (_tdot(ohT, a1) + _tdot(ohT, a2)) + _tdot(ohT, a3)

        mask_pos = mask[base:base + M, :]
        amask_ref[i] = jnp.sum(jnp.where(sel, mask_pos, 0.0),
                               axis=0, keepdims=True)


def kernel(fc_w1, fc_b1, fc_w2, fc_b2, proj_w1, proj_b1, proj_w2, proj_b2,
           gpn_obj_ind, gpn_pred_ind, gpn_nrel_ind, gpn_pool_mtx,
           att_feats, x_pred, fc_feats, att_masks):
    b, two, M, N, _ = gpn_pool_mtx.shape
    O, L = att_feats.shape[1], att_feats.shape[2]
    hid = fc_w1.shape[-1]
    G = two * b * M
    IB = _IB
    R = b * two * M
    B2 = IB * two * M

    pool_r = gpn_pool_mtx.reshape(R, N, N)
    ind_r = gpn_obj_ind.reshape(R, N)
    ind2_r = gpn_obj_ind.reshape(b, 1, two * M * N)
    mask_r = att_masks.reshape(R, N)

    scores, lparts, fc, af, amask = pl.pallas_call(
        _fused_kernel,
        out_shape=(
            jax.ShapeDtypeStruct((two, b, M, 1), jnp.float32),
            jax.ShapeDtypeStruct((b, 1, 1), jnp.float32),
            jax.ShapeDtypeStruct((b, 1, 2 * L), jnp.float32),
            jax.ShapeDtypeStruct((b, N, L), jnp.float32),
            jax.ShapeDtypeStruct((b, 1, N), jnp.float32),
        ),
        grid=(b // IB,),
        in_specs=[
            pl.BlockSpec((B2, N, N), lambda g: (g, 0, 0)),
            pl.BlockSpec((B2, N), lambda g: (g, 0)),
            pl.BlockSpec((IB, 1, two * M * N), lambda g: (g, 0, 0)),
            pl.BlockSpec((IB, O, L), lambda g: (g, 0, 0)),
            pl.BlockSpec((B2, N), lambda g: (g, 0)),
            pl.BlockSpec((2 * L, hid), lambda g: (0, 0)),
            pl.BlockSpec((1, hid), lambda g: (0, 0)),
            pl.BlockSpec((1, hid), lambda g: (0, 0)),
            pl.BlockSpec((1, 1), lambda g: (0, 0)),
            pl.BlockSpec((2 * L, hid), lambda g: (0, 0)),
            pl.BlockSpec((1, hid), lambda g: (0, 0)),
            pl.BlockSpec((hid, 2 * L), lambda g: (0, 0)),
            pl.BlockSpec((1, 2 * L), lambda g: (0, 0)),
        ],
        out_specs=(
            pl.BlockSpec((two, IB, M, 1), lambda g: (0, g, 0, 0)),
            pl.BlockSpec((IB, 1, 1), lambda g: (g, 0, 0)),
            pl.BlockSpec((IB, 1, 2 * L), lambda g: (g, 0, 0)),
            pl.BlockSpec((IB, N, L), lambda g: (g, 0, 0)),
            pl.BlockSpec((IB, 1, N), lambda g: (g, 0, 0)),
        ),
        compiler_params=pltpu.CompilerParams(
            dimension_semantics=("parallel",)),
    )(pool_r, ind_r, ind2_r, att_feats, mask_r,
      fc_w1, fc_b1, fc_w2, fc_b2, proj_w1, proj_b1, proj_w2, proj_b2)

    gpn_loss = jnp.sum(lparts) / G
    subgraph_score = scores.reshape(G, 1)
    fc_feats_out = fc.reshape(b, 2 * L)
    att_masks_out = amask.reshape(b, N)

    return gpn_loss, subgraph_score, af, fc_feats_out, att_masks_out


# speedup vs baseline: 1.2046x; 1.0606x over previous
import jax
import jax.numpy as jnp
from jax.experimental import pallas as pl
from jax.experimental.pallas import tpu as pltpu

_IB = 8


def _tdot(lhs, rhs):
    return jax.lax.dot_general(lhs, rhs, (((0,), (0,)), ((), ())),
                               preferred_element_type=jnp.float32)


def _fused_kernel(pool_ref, ind_ref, ind2_ref, att_ref, mask_ref,
                  fw1_ref, fb1_ref, fw2_ref, fb2_ref,
                  pw1_ref, pb1_ref, pw2_ref, pb2_ref,
                  score_ref, lsum_ref, fc_ref, af_ref, amask_ref):
    B2, N, _ = pool_ref.shape
    IB, O, L = att_ref.shape
    M = B2 // (2 * IB)
    S = 2 * M

    indb = ind_ref[...]
    att3 = att_ref[...]
    iota_f = jax.lax.broadcasted_iota(jnp.int32, (O, S * N), 0)
    g_list = []
    for i in range(IB):
        oh_i = jnp.where(ind2_ref[i] == iota_f, 1.0, 0.0)
        g_list.append(jax.lax.dot_general(
            oh_i, att3[i],
            dimension_numbers=(((0,), (0,)), ((), ())),
            preferred_element_type=jnp.float32))
    gathered = jnp.concatenate(g_list, axis=0).reshape(B2, N, L)

    clean = jax.lax.dot_general(
        pool_ref[...], gathered,
        dimension_numbers=(((2,), (1,)), ((0,), (0,))),
        preferred_element_type=jnp.float32)

    max_feat = jnp.max(clean, axis=1)
    mask = mask_ref[...]
    mask_sum = jnp.sum(mask, axis=1, keepdims=True)
    inv = pl.reciprocal(jnp.maximum(mask_sum, 1.0), approx=True)
    mean_feat = jnp.sum(clean, axis=1) * inv

    w1 = fw1_ref[...]
    h = (jnp.dot(max_feat, w1[:L, :], preferred_element_type=jnp.float32)
         + jnp.dot(mean_feat, w1[L:, :], preferred_element_type=jnp.float32)
         + fb1_ref[...])
    h = jnp.maximum(h, 0.0)
    z = jnp.sum(h * fw2_ref[...], axis=-1, keepdims=True) + fb2_ref[...]
    p = jax.nn.sigmoid(z)

    row = jax.lax.broadcasted_iota(jnp.int32, (B2, 1), 0)
    t = jnp.where(row % S < M, 1.0, 0.0)
    log_p = jnp.maximum(jnp.log(p), -100.0)
    log_1mp = jnp.maximum(jnp.log(1.0 - p), -100.0)
    lv = -(t * log_p + (1.0 - t) * log_1mp)

    g_id = pl.program_id(0)
    nsteps = pl.num_programs(0)
    G = nsteps * B2

    @pl.when(g_id == 0)
    def _():
        lsum_ref[...] = jnp.zeros_like(lsum_ref)

    lsum_ref[...] += jnp.sum(lv, keepdims=True)

    @pl.when(g_id == nsteps - 1)
    def _():
        lsum_ref[...] = lsum_ref[...] * (1.0 / G)

    m_idx = jax.lax.broadcasted_iota(jnp.int32, (M, 1), 0)
    iota_on = jax.lax.broadcasted_iota(jnp.int32, (O, N), 0)
    for i in range(IB):
        base = i * S
        score_ref[0, i] = p[base:base + M]
        score_ref[1, i] = p[base + M:base + S]

        p_pos = p[base:base + M]
        am = jnp.min(jnp.where(p_pos == jnp.max(p_pos), m_idx, M))
        sel = m_idx == am

        smax = jnp.sum(jnp.where(sel, max_feat[base:base + M, :], 0.0),
                       axis=0, keepdims=True)
        smean = jnp.sum(jnp.where(sel, mean_feat[base:base + M, :], 0.0),
                        axis=0, keepdims=True)
        x = jnp.concatenate([smax, smean], axis=1)
        hh = (jnp.dot(x, pw1_ref[...], preferred_element_type=jnp.float32)
              + pb1_ref[...])
        fc_ref[i] = (jnp.dot(hh, pw2_ref[...], preferred_element_type=jnp.float32)
                     + pb2_ref[...])

        ind_pos = indb[base:base + M, :]
        rowsel = jnp.sum(jnp.where(sel, ind_pos, 0), axis=0, keepdims=True)
        ohT = (iota_on == rowsel).astype(jnp.bfloat16)
        a = att3[i]
        a1 = a.astype(jnp.bfloat16)
        r1 = a - a1.astype(jnp.float32)
        a2 = r1.astype(jnp.bfloat16)
        a3 = (r1 - a2.astype(jnp.float32)).astype(jnp.bfloat16)
        af_ref[i] = (_tdot(ohT, a1) + _tdot(ohT, a2)) + _tdot(ohT, a3)

        mask_pos = mask[base:base + M, :]
        amask_ref[i] = jnp.sum(jnp.where(sel, mask_pos, 0.0),
                               axis=0, keepdims=True)


def kernel(fc_w1, fc_b1, fc_w2, fc_b2, proj_w1, proj_b1, proj_w2, proj_b2,
           gpn_obj_ind, gpn_pred_ind, gpn_nrel_ind, gpn_pool_mtx,
           att_feats, x_pred, fc_feats, att_masks):
    b, two, M, N, _ = gpn_pool_mtx.shape
    O, L = att_feats.shape[1], att_feats.shape[2]
    hid = fc_w1.shape[-1]
    G = two * b * M
    IB = _IB
    R = b * two * M
    B2 = IB * two * M

    pool_r = gpn_pool_mtx.reshape(R, N, N)
    ind_r = gpn_obj_ind.reshape(R, N)
    ind2_r = gpn_obj_ind.reshape(b, 1, two * M * N)
    mask_r = att_masks.reshape(R, N)

    scores, lsum, fc, af, amask = pl.pallas_call(
        _fused_kernel,
        out_shape=(
            jax.ShapeDtypeStruct((two, b, M, 1), jnp.float32),
            jax.ShapeDtypeStruct((1, 1), jnp.float32),
            jax.ShapeDtypeStruct((b, 1, 2 * L), jnp.float32),
            jax.ShapeDtypeStruct((b, N, L), jnp.float32),
            jax.ShapeDtypeStruct((b, 1, N), jnp.float32),
        ),
        grid=(b // IB,),
        in_specs=[
            pl.BlockSpec((B2, N, N), lambda g: (g, 0, 0)),
            pl.BlockSpec((B2, N), lambda g: (g, 0)),
            pl.BlockSpec((IB, 1, two * M * N), lambda g: (g, 0, 0)),
            pl.BlockSpec((IB, O, L), lambda g: (g, 0, 0)),
            pl.BlockSpec((B2, N), lambda g: (g, 0)),
            pl.BlockSpec((2 * L, hid), lambda g: (0, 0)),
            pl.BlockSpec((1, hid), lambda g: (0, 0)),
            pl.BlockSpec((1, hid), lambda g: (0, 0)),
            pl.BlockSpec((1, 1), lambda g: (0, 0)),
            pl.BlockSpec((2 * L, hid), lambda g: (0, 0)),
            pl.BlockSpec((1, hid), lambda g: (0, 0)),
            pl.BlockSpec((hid, 2 * L), lambda g: (0, 0)),
            pl.BlockSpec((1, 2 * L), lambda g: (0, 0)),
        ],
        out_specs=(
            pl.BlockSpec((two, IB, M, 1), lambda g: (0, g, 0, 0)),
            pl.BlockSpec((1, 1), lambda g: (0, 0)),
            pl.BlockSpec((IB, 1, 2 * L), lambda g: (g, 0, 0)),
            pl.BlockSpec((IB, N, L), lambda g: (g, 0, 0)),
            pl.BlockSpec((IB, 1, N), lambda g: (g, 0, 0)),
        ),
        compiler_params=pltpu.CompilerParams(
            dimension_semantics=("arbitrary",)),
    )(pool_r, ind_r, ind2_r, att_feats, mask_r,
      fc_w1, fc_b1, fc_w2, fc_b2, proj_w1, proj_b1, proj_w2, proj_b2)

    gpn_loss = lsum.reshape(())
    subgraph_score = scores.reshape(G, 1)
    fc_feats_out = fc.reshape(b, 2 * L)
    att_masks_out = amask.reshape(b, N)

    return gpn_loss, subgraph_score, af, fc_feats_out, att_masks_out
